# Initial kernel scaffold; baseline (speedup 1.0000x reference)
import jax
import jax.numpy as jnp
from jax.experimental import pallas as pl


def kernel(x, W_gate, b_gate, weight1, weight2):
    raise NotImplementedError("write your pallas kernel here")


# speedup vs baseline: 11.6617x; 11.6617x over previous
import jax
import jax.numpy as jnp
from jax.experimental import pallas as pl

_B, _S, _D = 4, 8192, 128
_E, _K, _H, _O = 64, 8, 512, 128
_NEG = float("-inf")


def _gelu(h):
    return 0.5 * h * (1.0 + jax.lax.erf(h * 0.7071067811865476))


def _gating_body(x_ref, wg_ref, bg_ref, p_ref, inp_ref):
    x = x_ref[0]
    wg = wg_ref[...]
    logits = jax.lax.dot_general(
        wg, x, (((1,), (1,)), ((), ())), preferred_element_type=jnp.float32
    )
    logits = logits + bg_ref[...]
    m = jnp.max(logits, axis=1, keepdims=True)
    el = jnp.exp(logits - m)
    z = jnp.sum(el, axis=1, keepdims=True)
    iota = jax.lax.broadcasted_iota(jnp.int32, (_E, _S), 1)
    lw = logits
    msk = jnp.zeros((_E, _S), jnp.float32)
    for _ in range(_K):
        cm = jnp.max(lw, axis=1, keepdims=True)
        cidx = jnp.min(jnp.where(lw == cm, iota, _S), axis=1, keepdims=True)
        hit = iota == cidx
        msk = jnp.where(hit, 1.0, msk)
        lw = jnp.where(hit, _NEG, lw)
    p = msk * el / z
    p_ref[0] = p
    inp_ref[0] = jax.lax.dot_general(
        p, x, (((1,), (0,)), ((), ())), preferred_element_type=jnp.float32
    )


_EB = 8


def _ffn_body(inp_ref, w1_ref, w2_ref, out_ref):
    for e in range(_EB):
        v = inp_ref[:, e, :]
        w1 = w1_ref[e]
        h = jnp.dot(v, w1[:_D], preferred_element_type=jnp.float32) + w1[_D:_D + 1]
        h = _gelu(h)
        w2 = w2_ref[e]
        out_ref[:, e, :] = (
            jnp.dot(h, w2[:_H], preferred_element_type=jnp.float32) + w2[_H:_H + 1]
        )


def _combine_body(p_ref, out_ref, y_ref):
    p = p_ref[0]
    o = out_ref[0]
    y_ref[0] = jax.lax.dot_general(
        p, o, (((0,), (0,)), ((), ())), preferred_element_type=jnp.float32
    )


def kernel(x, W_gate, b_gate, weight1, weight2):
    bg2 = b_gate.reshape(_E, 1)
    p, inp = pl.pallas_call(
        _gating_body,
        grid=(_B,),
        in_specs=[
            pl.BlockSpec((1, _S, _D), lambda b: (b, 0, 0)),
            pl.BlockSpec((_E, _D), lambda b: (0, 0)),
            pl.BlockSpec((_E, 1), lambda b: (0, 0)),
        ],
        out_specs=[
            pl.BlockSpec((1, _E, _S), lambda b: (b, 0, 0)),
            pl.BlockSpec((1, _E, _D), lambda b: (b, 0, 0)),
        ],
        out_shape=[
            jax.ShapeDtypeStruct((_B, _E, _S), jnp.float32),
            jax.ShapeDtypeStruct((_B, _E, _D), jnp.float32),
        ],
    )(x, W_gate, bg2)

    out = pl.pallas_call(
        _ffn_body,
        grid=(_E // _EB,),
        in_specs=[
            pl.BlockSpec((_B, _EB, _D), lambda e: (0, e, 0)),
            pl.BlockSpec((_EB, _D + 1, _H), lambda e: (e, 0, 0)),
            pl.BlockSpec((_EB, _H + 1, _O), lambda e: (e, 0, 0)),
        ],
        out_specs=pl.BlockSpec((_B, _EB, _O), lambda e: (0, e, 0)),
        out_shape=jax.ShapeDtypeStruct((_B, _E, _O), jnp.float32),
    )(inp, weight1, weight2)

    y = pl.pallas_call(
        _combine_body,
        grid=(_B,),
        in_specs=[
            pl.BlockSpec((1, _E, _S), lambda b: (b, 0, 0)),
            pl.BlockSpec((1, _E, _O), lambda b: (b, 0, 0)),
        ],
        out_specs=pl.BlockSpec((1, _S, _O), lambda b: (b, 0, 0)),
        out_shape=jax.ShapeDtypeStruct((_B, _S, _O), jnp.float32),
    )(p, out)
    return y
